# Initial kernel scaffold; baseline (speedup 1.0000x reference)
import jax
import jax.numpy as jnp
from jax.experimental import pallas as pl


def kernel(z, emb_w):
    raise NotImplementedError("write your pallas kernel here")


# speedup vs baseline: 7.0744x; 7.0744x over previous
import jax
import jax.numpy as jnp
from jax.experimental import pallas as pl
from jax.experimental.pallas import tpu as pltpu

N_E = 8192
E_DIM = 32
N_TOK = 8192
BETA = 0.25

RA = 1024
CA = 1024


def _argmin_body(z_ref, emb_ref, idx_ref, bestv, besti):
    c = pl.program_id(1)
    nc = pl.num_programs(1)

    @pl.when(c == 0)
    def _():
        bestv[...] = jnp.full_like(bestv, jnp.inf)
        besti[...] = jnp.zeros_like(besti)

    zb = z_ref[...]
    eb = emb_ref[...]
    m = jax.lax.dot_general(zb.astype(jnp.bfloat16), eb, (((1,), (1,)), ((), ())),
                            preferred_element_type=jnp.float32)
    s1 = jnp.sum(zb * zb, axis=1, keepdims=True)
    s2 = jnp.sum(eb * eb, axis=1)
    d = (s1 + s2[None, :]) - 2.0 * m
    bmin = jnp.min(d, axis=1, keepdims=True)
    ids = jax.lax.broadcasted_iota(jnp.int32, (RA, CA), 1)
    big = jnp.int32(2**31 - 1)
    bidx = jnp.min(jnp.where(d == bmin, ids, big), axis=1, keepdims=True) + c * CA
    upd = bmin < bestv[...]
    besti[...] = jnp.where(upd, bidx, besti[...])
    bestv[...] = jnp.where(upd, bmin, bestv[...])

    @pl.when(c == nc - 1)
    def _():
        idx_ref[...] = besti[...]


def _argmin_call(z_flat, emb_w):
    grid = (N_TOK // RA, N_E // CA)
    return pl.pallas_call(
        _argmin_body,
        grid=grid,
        in_specs=[
            pl.BlockSpec((RA, E_DIM), lambda r, c: (r, 0)),
            pl.BlockSpec((CA, E_DIM), lambda r, c: (c, 0)),
        ],
        out_specs=pl.BlockSpec((RA, 1), lambda r, c: (r, 0)),
        out_shape=jax.ShapeDtypeStruct((N_TOK, 1), jnp.int32),
        scratch_shapes=[
            pltpu.VMEM((RA, 1), jnp.float32),
            pltpu.VMEM((RA, 1), jnp.int32),
        ],
    )(z_flat, emb_w)


RB = 256


def _onehot_body(idx_ref, z_ref, emb_ref,
                 enc_ref, zq_ref, hist_ref, loss_ref, perp_ref):
    r = pl.program_id(0)
    nr = pl.num_programs(0)

    @pl.when(r == 0)
    def _():
        hist_ref[...] = jnp.zeros_like(hist_ref)
        loss_ref[0, 0] = 0.0

    idx = idx_ref[...]
    ids = jax.lax.broadcasted_iota(jnp.int32, (RB, N_E), 1)
    oh = jnp.where(ids == idx, 1.0, 0.0).astype(jnp.float32)
    enc_ref[...] = oh
    zq = jax.lax.dot_general(oh, emb_ref[...], (((1,), (0,)), ((), ())),
                             preferred_element_type=jnp.float32)
    hist_ref[...] += jnp.sum(oh, axis=0).reshape(N_E // 1024, 1024)
    zb = z_ref[...]
    zq_ref[...] = zb + (zq - zb)
    loss_ref[0, 0] += jnp.sum((zq - zb) ** 2)

    @pl.when(r == nr - 1)
    def _():
        loss_ref[0, 0] = loss_ref[0, 0] * ((1.0 + BETA) / (N_TOK * E_DIM))
        em = hist_ref[...] * (1.0 / N_TOK)
        perp_ref[0, 0] = jnp.exp(-jnp.sum(em * jnp.log(em + 1e-10)))


def _onehot_call(idx, z_flat, emb_w):
    grid = (N_TOK // RB,)
    return pl.pallas_call(
        _onehot_body,
        grid=grid,
        in_specs=[
            pl.BlockSpec((RB, 1), lambda r: (r, 0)),
            pl.BlockSpec((RB, E_DIM), lambda r: (r, 0)),
            pl.BlockSpec((N_E, E_DIM), lambda r: (0, 0)),
        ],
        out_specs=[
            pl.BlockSpec((RB, N_E), lambda r: (r, 0)),
            pl.BlockSpec((RB, E_DIM), lambda r: (r, 0)),
            pl.BlockSpec((N_E // 1024, 1024), lambda r: (0, 0)),
            pl.BlockSpec(memory_space=pltpu.SMEM),
            pl.BlockSpec(memory_space=pltpu.SMEM),
        ],
        out_shape=[
            jax.ShapeDtypeStruct((N_TOK, N_E), jnp.float32),
            jax.ShapeDtypeStruct((N_TOK, E_DIM), jnp.float32),
            jax.ShapeDtypeStruct((N_E // 1024, 1024), jnp.float32),
            jax.ShapeDtypeStruct((1, 1), jnp.float32),
            jax.ShapeDtypeStruct((1, 1), jnp.float32),
        ],
    )(idx, z_flat, emb_w)


def kernel(z, emb_w):
    zp = jnp.transpose(z, (0, 2, 3, 1))
    z_flat = zp.reshape(-1, E_DIM)
    idx = _argmin_call(z_flat, emb_w)
    enc, zq_flat, _hist, loss, perp = _onehot_call(idx, z_flat, emb_w)
    z_q = zq_flat.reshape(zp.shape).transpose(0, 3, 1, 2)
    return (loss[0, 0], z_q, perp[0, 0], enc, idx)
